# baseline (speedup 1.0000x reference)
import functools

import jax
import jax.numpy as jnp
from jax import lax
from jax.experimental import pallas as pl
from jax.experimental.pallas import tpu as pltpu
from jax.experimental.pallas import tpu_sc as plsc

NUM_EMB = 1_000_000
EMBEDDING_DIM = 32
BATCH = 16384
TC_BATCH = 6144
SC_BATCH = BATCH - TC_BATCH
TC_PER_STEP = 8
TC_STEPS = TC_BATCH // TC_PER_STEP
NUM_CORES = 2
NUM_SUBCORES = 16
NUM_WORKERS = NUM_CORES * NUM_SUBCORES
B_PER_W = SC_BATCH // NUM_WORKERS
LANES = 16
CHUNK = 8
NPAIR = B_PER_W // (2 * CHUNK)
MAX_COL = (NUM_EMB // 128) * 128 - 128
TAIL_START = (NUM_EMB // 128) * 128


@functools.partial(
    pl.kernel,
    mesh=plsc.VectorSubcoreMesh(core_axis_name="c", subcore_axis_name="s"),
    out_type=jax.ShapeDtypeStruct((SC_BATCH, EMBEDDING_DIM), jnp.float32),
    scratch_types=[
        pltpu.VMEM((B_PER_W,), jnp.int32),
        pltpu.VMEM((CHUNK, EMBEDDING_DIM, 128), jnp.float32),
        pltpu.VMEM((CHUNK, EMBEDDING_DIM, 128), jnp.float32),
        pltpu.VMEM((CHUNK, EMBEDDING_DIM), jnp.float32),
        pltpu.SemaphoreType.DMA,
        pltpu.SemaphoreType.DMA,
    ],
    compiler_params=pltpu.CompilerParams(
        use_tc_tiling_on_sc=True, needs_layout_passes=False
    ),
)
def _emb_sc(wt, idx_hbm, out, idx_v, buf0, buf1, rows_v, sem0, sem1):
    wid = lax.axis_index("s") * NUM_CORES + lax.axis_index("c")
    base = wid * B_PER_W
    pltpu.sync_copy(idx_hbm.at[pl.ds(base, B_PER_W)], idx_v)
    bufs = (buf0, buf1)
    sems = (sem0, sem1)

    def pair_cols_lanes(p):
        ivec = idx_v[pl.ds(p * 2 * CHUNK, LANES)]
        cols = jnp.minimum(
            lax.shift_right_logical(ivec, 7) * 128,
            jnp.full((LANES,), MAX_COL, jnp.int32),
        )
        lanes_vec = jnp.minimum(ivec - cols, jnp.full((LANES,), 127, jnp.int32))
        return cols, lanes_vec

    def fire(cols, half, buf, sem):
        for kk in range(CHUNK):
            col = pl.multiple_of(cols[half * CHUNK + kk], 128)
            pltpu.async_copy(wt.at[:, pl.ds(col, 128)], buf.at[kk], sem)

    def drain(buf, sem):
        for kk in range(CHUNK):
            pltpu.make_async_copy(
                wt.at[:, pl.ds(0, 128)], buf.at[kk], sem
            ).wait()

    def extract(lanes_vec, half, buf):
        for kk in range(CHUNK):
            lane = lanes_vec[half * CHUNK + kk]
            for dd in range(EMBEDDING_DIM // LANES):
                rows = lax.iota(jnp.int32, LANES) + dd * LANES
                vals = plsc.load_gather(
                    buf,
                    [jnp.full((LANES,), kk, jnp.int32),
                     rows,
                     jnp.broadcast_to(lane, (LANES,))],
                )
                rows_v[kk, pl.ds(dd * LANES, LANES)] = vals

    cols0, _ = pair_cols_lanes(0)
    fire(cols0, 0, bufs[0], sems[0])
    fire(cols0, 1, bufs[1], sems[1])

    def body(p, carry):
        _, lanes_vec = pair_cols_lanes(p)
        ncols, _ = pair_cols_lanes(jnp.minimum(p + 1, NPAIR - 1))
        for half in range(2):
            drain(bufs[half], sems[half])
            extract(lanes_vec, half, bufs[half])
            c = p * 2 + half
            off = pl.multiple_of(base + c * CHUNK, 8)
            pltpu.sync_copy(rows_v, out.at[pl.ds(off, CHUNK)])

            @pl.when(p + 1 < NPAIR)
            def _():
                fire(ncols, half, bufs[half], sems[half])
        return carry

    lax.fori_loop(0, NPAIR, body, 0)


def _tc_body(idx_ref, *refs):
    blocks = refs[:TC_PER_STEP]
    out_ref = refs[TC_PER_STEP]
    step = pl.program_id(0)
    lane_iota = lax.broadcasted_iota(jnp.int32, (1, 128), 1)
    for j in range(TC_PER_STEP):
        i = idx_ref[step * TC_PER_STEP + j]
        lane = lax.bitwise_and(i, 127)
        sel = jnp.where(lane_iota == lane, 1.0, 0.0)
        out_ref[j, :] = jnp.sum(blocks[j][...] * sel, axis=1)


def _tc_block_spec(j):
    def index_map(step, idx_ref):
        c = lax.shift_right_logical(idx_ref[step * TC_PER_STEP + j], 7)
        return (0, jnp.minimum(c, NUM_EMB // 128 - 1))
    return pl.BlockSpec((EMBEDDING_DIM, 128), index_map)


_emb_tc = pl.pallas_call(
    _tc_body,
    grid_spec=pltpu.PrefetchScalarGridSpec(
        num_scalar_prefetch=1,
        grid=(TC_STEPS,),
        in_specs=[_tc_block_spec(j) for j in range(TC_PER_STEP)],
        out_specs=pl.BlockSpec(
            (TC_PER_STEP, EMBEDDING_DIM), lambda step, idx_ref: (step, 0)
        ),
    ),
    out_shape=jax.ShapeDtypeStruct((TC_BATCH, EMBEDDING_DIM), jnp.float32),
)


def kernel(x, weight):
    xi = x.astype(jnp.int32)
    wt = weight.T
    sc_out = _emb_sc(wt, xi[TC_BATCH:])
    tc_out = _emb_tc(xi[:TC_BATCH], *([wt] * TC_PER_STEP))
    main = jnp.concatenate([tc_out, sc_out], axis=0)
    tail_ids = jnp.clip(xi - TAIL_START, 0, NUM_EMB - TAIL_START - 1)
    tail = jnp.take(weight[TAIL_START:], tail_ids, axis=0)
    return jnp.where((xi >= TAIL_START)[:, None], tail, main)


# speedup vs baseline: 3.8350x; 3.8350x over previous
import functools

import jax
import jax.numpy as jnp
from jax import lax
from jax.experimental import pallas as pl
from jax.experimental.pallas import tpu as pltpu
from jax.experimental.pallas import tpu_sc as plsc

NUM_EMB = 1_000_000
EMBEDDING_DIM = 32
BATCH = 16384
NUM_CORES = 2
NUM_SUBCORES = 16
NUM_WORKERS = NUM_CORES * NUM_SUBCORES
B_PER_W = BATCH // NUM_WORKERS
LANES = 16
CHUNK = 8
NPAIR = B_PER_W // (2 * CHUNK)


@functools.partial(
    pl.kernel,
    mesh=plsc.VectorSubcoreMesh(core_axis_name="c", subcore_axis_name="s"),
    out_type=jax.ShapeDtypeStruct((BATCH, EMBEDDING_DIM), jnp.float32),
    scratch_types=[
        pltpu.VMEM((B_PER_W,), jnp.int32),
        pltpu.VMEM((CHUNK, EMBEDDING_DIM, 128), jnp.float32),
        pltpu.VMEM((CHUNK, EMBEDDING_DIM, 128), jnp.float32),
        pltpu.VMEM((CHUNK, EMBEDDING_DIM), jnp.float32),
        pltpu.SemaphoreType.DMA,
        pltpu.SemaphoreType.DMA,
    ],
    compiler_params=pltpu.CompilerParams(
        use_tc_tiling_on_sc=True, needs_layout_passes=False
    ),
)
def _emb_lookup(wt, idx_hbm, out, idx_v, buf0, buf1, rows_v, sem0, sem1):
    wid = lax.axis_index("s") * NUM_CORES + lax.axis_index("c")
    base = wid * B_PER_W
    pltpu.sync_copy(idx_hbm.at[pl.ds(base, B_PER_W)], idx_v)
    bufs = (buf0, buf1)
    sems = (sem0, sem1)

    def pair_cols_lanes(p):
        ivec = idx_v[pl.ds(p * 2 * CHUNK, LANES)]
        cols = lax.shift_right_logical(ivec, 7) * 128
        lanes_vec = lax.bitwise_and(ivec, 127)
        return cols, lanes_vec

    def fire(cols, half, buf, sem):
        for kk in range(CHUNK):
            col = pl.multiple_of(cols[half * CHUNK + kk], 128)
            pltpu.async_copy(wt.at[:, pl.ds(col, 128)], buf.at[kk], sem)

    def drain(buf, sem):
        for kk in range(CHUNK):
            pltpu.make_async_copy(
                wt.at[:, pl.ds(0, 128)], buf.at[kk], sem
            ).wait()

    def extract(lanes_vec, half, buf):
        for kk in range(CHUNK):
            lane = lanes_vec[half * CHUNK + kk]
            for dd in range(EMBEDDING_DIM // LANES):
                rows = lax.iota(jnp.int32, LANES) + dd * LANES
                vals = plsc.load_gather(
                    buf,
                    [jnp.full((LANES,), kk, jnp.int32),
                     rows,
                     jnp.broadcast_to(lane, (LANES,))],
                )
                rows_v[kk, pl.ds(dd * LANES, LANES)] = vals

    cols0, _ = pair_cols_lanes(0)
    fire(cols0, 0, bufs[0], sems[0])
    fire(cols0, 1, bufs[1], sems[1])

    def body(p, carry):
        _, lanes_vec = pair_cols_lanes(p)
        ncols, _ = pair_cols_lanes(jnp.minimum(p + 1, NPAIR - 1))
        for half in range(2):
            drain(bufs[half], sems[half])
            extract(lanes_vec, half, bufs[half])
            c = p * 2 + half
            off = pl.multiple_of(base + c * CHUNK, 8)
            pltpu.sync_copy(rows_v, out.at[pl.ds(off, CHUNK)])

            @pl.when(p + 1 < NPAIR)
            def _():
                fire(ncols, half, bufs[half], sems[half])
        return carry

    lax.fori_loop(0, NPAIR, body, 0)


def kernel(x, weight):
    return _emb_lookup(weight.T, x.astype(jnp.int32))


# speedup vs baseline: 4.1685x; 1.0870x over previous
import functools

import jax
import jax.numpy as jnp
from jax import lax
from jax.experimental import pallas as pl
from jax.experimental.pallas import tpu as pltpu
from jax.experimental.pallas import tpu_sc as plsc

NUM_EMB = 1_000_000
EMBEDDING_DIM = 32
BATCH = 16384
NUM_CORES = 2
NUM_SUBCORES = 16
NUM_WORKERS = NUM_CORES * NUM_SUBCORES
B_PER_W = BATCH // NUM_WORKERS
LANES = 16
CHUNK = 8
NRING = 3
NCHUNK = B_PER_W // CHUNK
NTRIP = NCHUNK // NRING
NTAIL = NCHUNK - NTRIP * NRING


@functools.partial(
    pl.kernel,
    mesh=plsc.VectorSubcoreMesh(core_axis_name="c", subcore_axis_name="s"),
    out_type=jax.ShapeDtypeStruct((BATCH, EMBEDDING_DIM), jnp.float32),
    scratch_types=[
        pltpu.VMEM((B_PER_W + LANES,), jnp.int32),
        pltpu.VMEM((CHUNK, EMBEDDING_DIM, 128), jnp.float32),
        pltpu.VMEM((CHUNK, EMBEDDING_DIM, 128), jnp.float32),
        pltpu.VMEM((CHUNK, EMBEDDING_DIM, 128), jnp.float32),
        pltpu.VMEM((CHUNK, EMBEDDING_DIM), jnp.float32),
        pltpu.SemaphoreType.DMA,
        pltpu.SemaphoreType.DMA,
        pltpu.SemaphoreType.DMA,
    ],
    compiler_params=pltpu.CompilerParams(
        use_tc_tiling_on_sc=True, needs_layout_passes=False
    ),
)
def _emb_lookup(wt, idx_hbm, out, idx_v, buf0, buf1, buf2, rows_v, sem0, sem1, sem2):
    wid = lax.axis_index("s") * NUM_CORES + lax.axis_index("c")
    base = wid * B_PER_W
    pltpu.sync_copy(idx_hbm.at[pl.ds(base, B_PER_W)], idx_v.at[pl.ds(0, B_PER_W)])
    bufs = (buf0, buf1, buf2)
    sems = (sem0, sem1, sem2)

    def chunk_cols_lanes(c):
        ivec = idx_v[pl.ds(c * CHUNK, LANES)]
        cols = lax.shift_right_logical(ivec, 7) * 128
        lanes_vec = lax.bitwise_and(ivec, 127)
        return cols, lanes_vec

    def fire(cols, half, buf, sem):
        for kk in range(CHUNK):
            col = pl.multiple_of(cols[half * CHUNK + kk], 128)
            pltpu.async_copy(wt.at[:, pl.ds(col, 128)], buf.at[kk], sem)

    def fire_chunk(c, buf, sem):
        cols, _ = chunk_cols_lanes(c)
        fire(cols, 0, buf, sem)

    def drain(buf, sem):
        for kk in range(CHUNK):
            pltpu.make_async_copy(
                wt.at[:, pl.ds(0, 128)], buf.at[kk], sem
            ).wait()

    def extract(lanes_vec, half, buf):
        for kk in range(CHUNK):
            lane = lanes_vec[half * CHUNK + kk]
            for dd in range(EMBEDDING_DIM // LANES):
                rows = lax.iota(jnp.int32, LANES) + dd * LANES
                vals = plsc.load_gather(
                    buf,
                    [jnp.full((LANES,), kk, jnp.int32),
                     rows,
                     jnp.broadcast_to(lane, (LANES,))],
                )
                rows_v[kk, pl.ds(dd * LANES, LANES)] = vals

    for r in range(NRING):
        fire_chunk(r, bufs[r], sems[r])

    def process(c, buf, sem, refire_c):
        _, lanes_vec = chunk_cols_lanes(c)
        drain(buf, sem)
        extract(lanes_vec, 0, buf)
        off = pl.multiple_of(base + c * CHUNK, 8)
        pltpu.sync_copy(rows_v, out.at[pl.ds(off, CHUNK)])

        @pl.when(refire_c < NCHUNK)
        def _():
            fire_chunk(refire_c, buf, sem)

    def body(p, carry):
        for r in range(NRING):
            c = p * NRING + r
            process(c, bufs[r], sems[r], c + NRING)
        return carry

    lax.fori_loop(0, NTRIP, body, 0)
    for r in range(NTAIL):
        c = NTRIP * NRING + r
        process(c, bufs[r], sems[r], jnp.int32(NCHUNK))


def kernel(x, weight):
    return _emb_lookup(weight.T, x.astype(jnp.int32))
